# Initial kernel scaffold; baseline (speedup 1.0000x reference)
import jax
import jax.numpy as jnp
from jax.experimental import pallas as pl


def kernel(query, mem_keys, mem_values, cW1, cb1, cW2, cb2, cW3, cb3, dW1, db1, dW2, db2, dW3, db3, k):
    raise NotImplementedError("write your pallas kernel here")


# speedup vs baseline: 1.5298x; 1.5298x over previous
import functools

import jax
import jax.numpy as jnp
from jax import lax
from jax.experimental import pallas as pl
from jax.experimental.pallas import tpu as pltpu
from jax.experimental.pallas import tpu_sc as plsc

KSEL = 16


def _mlp3_relu(x, W1, b1, W2, b2, W3, b3):
    h = jnp.maximum(x @ W1 + b1, 0.0)
    h = jnp.maximum(h @ W2 + b2, 0.0)
    return h @ W3 + b3


def _extract_topk(d, gidx, big_idx, n):
    vs, js = [], []
    for _ in range(n):
        m = jnp.min(d, axis=1, keepdims=True)
        sel = jnp.min(jnp.where(d == m, gidx, big_idx), axis=1, keepdims=True)
        vs.append(m)
        js.append(sel)
        d = jnp.where(gidx == sel, jnp.inf, d)
    return jnp.concatenate(vs, axis=1), jnp.concatenate(js, axis=1)


def _topk_body(num_tiles, kt,
               q_ref, keys_ref, W1r, b1r, W2r, b2r, W3r, b3r,
               idx_out, w_out, conf_out,
               cq_s, qn_s, tv_s, ti_s):
    t = pl.program_id(0)

    @pl.when(t == 0)
    def _init():
        cq = _mlp3_relu(q_ref[...], W1r[...], b1r[...], W2r[...], b2r[...],
                        W3r[...], b3r[...])
        cq_s[...] = cq
        qn_s[...] = jnp.sum(cq * cq, axis=1, keepdims=True)
        tv_s[...] = jnp.full(tv_s.shape, jnp.inf, jnp.float32)
        ti_s[...] = jnp.zeros(ti_s.shape, jnp.int32)

    ck = _mlp3_relu(keys_ref[...], W1r[...], b1r[...], W2r[...], b2r[...],
                    W3r[...], b3r[...])
    kn = jnp.sum(ck * ck, axis=1)
    qk = lax.dot_general(cq_s[...], ck, (((1,), (1,)), ((), ())),
                         preferred_element_type=jnp.float32)
    d = qn_s[...] + kn[None, :] - 2.0 * qk

    big_idx = jnp.int32(2 ** 30)
    gidx = (lax.broadcasted_iota(jnp.int32, d.shape, 1)
            + t * jnp.int32(kt))
    cv, ci = _extract_topk(d, gidx, big_idx, KSEL)

    allv = jnp.concatenate([tv_s[...], cv], axis=1)
    alli = jnp.concatenate([ti_s[...], ci], axis=1)
    nv, ni = _extract_topk(allv, alli, big_idx, KSEL)
    tv_s[...] = nv
    ti_s[...] = ni

    w = 1.0 / (nv + 1e-6)
    w_out[...] = w / jnp.sum(w, axis=1, keepdims=True)
    idx_out[...] = ni
    conf_out[...] = 1.0 / (nv[:, :1] + 1e-6)


def _topk_search(query, mem_keys, cW1, cb1, cW2, cb2, cW3, cb3, kt):
    Q, D = query.shape
    K = mem_keys.shape[0]
    C = cW3.shape[1]
    assert K % kt == 0
    num_tiles = K // kt
    const = lambda *_: (0, 0)
    grid_spec = pltpu.PrefetchScalarGridSpec(
        num_scalar_prefetch=0,
        grid=(num_tiles,),
        in_specs=[
            pl.BlockSpec((Q, D), const),
            pl.BlockSpec((kt, D), lambda t: (t, 0)),
            pl.BlockSpec(cW1.shape, const),
            pl.BlockSpec((1, cb1.shape[-1]), const),
            pl.BlockSpec(cW2.shape, const),
            pl.BlockSpec((1, cb2.shape[-1]), const),
            pl.BlockSpec(cW3.shape, const),
            pl.BlockSpec((1, cb3.shape[-1]), const),
        ],
        out_specs=[
            pl.BlockSpec((Q, KSEL), const),
            pl.BlockSpec((Q, KSEL), const),
            pl.BlockSpec((Q, 1), const),
        ],
        scratch_shapes=[
            pltpu.VMEM((Q, C), jnp.float32),
            pltpu.VMEM((Q, 1), jnp.float32),
            pltpu.VMEM((Q, KSEL), jnp.float32),
            pltpu.VMEM((Q, KSEL), jnp.int32),
        ],
    )
    return pl.pallas_call(
        functools.partial(_topk_body, num_tiles, kt),
        grid_spec=grid_spec,
        out_shape=[
            jax.ShapeDtypeStruct((Q, KSEL), jnp.int32),
            jax.ShapeDtypeStruct((Q, KSEL), jnp.float32),
            jax.ShapeDtypeStruct((Q, 1), jnp.float32),
        ],
        compiler_params=pltpu.CompilerParams(
            dimension_semantics=("arbitrary",)),
    )(query, mem_keys, cW1, cb1.reshape(1, -1), cW2, cb2.reshape(1, -1),
      cW3, cb3.reshape(1, -1))


def _sc_gather(table, idx2d):
    V, C = table.shape
    R, CH = idx2d.shape
    B = R * CH
    info = plsc.get_sparse_core_info()
    NW = info.num_cores * info.num_subcores
    rows_per_w = R // NW
    b_per_w = rows_per_w * CH
    mesh = plsc.VectorSubcoreMesh(core_axis_name="c", subcore_axis_name="s")

    @functools.partial(
        pl.kernel, mesh=mesh,
        out_type=jax.ShapeDtypeStruct((B, C), jnp.float32),
        scratch_types=[
            pltpu.VMEM((rows_per_w, CH), jnp.int32),
            pltpu.VMEM((b_per_w, C), jnp.float32),
            pltpu.SemaphoreType.DMA,
        ],
    )
    def kfn(table_hbm, idx_hbm, out_hbm, idx_v, rows_v, sem):
        wid = lax.axis_index("s") * info.num_cores + lax.axis_index("c")
        pltpu.sync_copy(idx_hbm.at[pl.ds(wid * rows_per_w, rows_per_w)], idx_v)
        copies = []
        for c in range(rows_per_w):
            copies.append(pltpu.async_copy(
                table_hbm.at[idx_v.at[c]],
                rows_v.at[pl.ds(c * CH, CH)], sem))
        for cp in copies:
            cp.wait()
        pltpu.sync_copy(rows_v, out_hbm.at[pl.ds(wid * b_per_w, b_per_w)])

    return kfn(table, idx2d)


def _combine_body(g_ref, w_ref, W1r, b1r, W2r, b2r, W3r, b3r, out_ref):
    w = w_ref[...]
    acc = g_ref[:, 0, :] * w[:, 0:1]
    for j in range(1, KSEL):
        acc = acc + g_ref[:, j, :] * w[:, j:j + 1]
    out_ref[...] = _mlp3_relu(acc, W1r[...], b1r[...], W2r[...], b2r[...],
                              W3r[...], b3r[...])


def _combine_decompress(gathered3, weights, dW1, db1, dW2, db2, dW3, db3):
    Q, ks, C = gathered3.shape
    D = dW3.shape[1]
    return pl.pallas_call(
        _combine_body,
        out_shape=jax.ShapeDtypeStruct((Q, D), jnp.float32),
    )(gathered3, weights, dW1, db1.reshape(1, -1), dW2, db2.reshape(1, -1),
      dW3, db3.reshape(1, -1))


def kernel(query, mem_keys, mem_values,
           cW1, cb1, cW2, cb2, cW3, cb3,
           dW1, db1, dW2, db2, dW3, db3, k):
    Q = query.shape[0]
    C = mem_values.shape[1]
    idx, weights, conf = _topk_search(query, mem_keys,
                                      cW1, cb1, cW2, cb2, cW3, cb3, kt=2000)
    idx2d = idx.reshape(-1, 128)
    gathered = _sc_gather(mem_values, idx2d)
    retrieved = _combine_decompress(gathered.reshape(Q, KSEL, C), weights,
                                    dW1, db1, dW2, db2, dW3, db3)
    return retrieved, conf.reshape(Q)


# speedup vs baseline: 1.8296x; 1.1960x over previous
import functools

import jax
import jax.numpy as jnp
from jax import lax
from jax.experimental import pallas as pl
from jax.experimental.pallas import tpu as pltpu
from jax.experimental.pallas import tpu_sc as plsc

KSEL = 16


def _mlp3_relu(x, W1, b1, W2, b2, W3, b3):
    h = jnp.maximum(x @ W1 + b1, 0.0)
    h = jnp.maximum(h @ W2 + b2, 0.0)
    return h @ W3 + b3


def _stream_merge(dbuf, col0, tv_s, ti_s):
    Q, kt = dbuf.shape
    big_idx = jnp.int32(2 ** 30)

    def body(carry):
        d = dbuf[...]
        gidx = lax.broadcasted_iota(jnp.int32, (Q, kt), 1) + col0
        m = jnp.min(d, axis=1, keepdims=True)
        sel = jnp.min(jnp.where(d == m, gidx, big_idx), axis=1,
                      keepdims=True)
        dbuf[...] = jnp.where(gidx == sel, jnp.inf, d)
        tv = tv_s[...]
        ti = ti_s[...]
        thr = tv[:, KSEL - 1:KSEL]
        tvm1 = jnp.concatenate(
            [jnp.full((Q, 1), -jnp.inf, jnp.float32), tv[:, :KSEL - 1]], 1)
        tim1 = jnp.concatenate(
            [jnp.zeros((Q, 1), jnp.int32), ti[:, :KSEL - 1]], 1)
        stay = tv <= m
        ins = jnp.logical_and(jnp.logical_not(stay), tvm1 <= m)
        tv_s[...] = jnp.where(stay, tv, jnp.where(ins, m, tvm1))
        ti_s[...] = jnp.where(stay, ti, jnp.where(ins, sel, tim1))
        return jnp.any(m < thr)

    lax.fori_loop(0, KSEL + 1, lambda i, c: body(c), jnp.bool_(True))


def _topk_body(num_tiles, kt,
               q_ref, keys_ref, W1r, b1r, W2r, b2r, W3r, b3r,
               idx_out, w_out, conf_out,
               cq_s, qn_s, tv_s, ti_s, db_s):
    t = pl.program_id(0)

    @pl.when(t == 0)
    def _init():
        cq = _mlp3_relu(q_ref[...], W1r[...], b1r[...], W2r[...], b2r[...],
                        W3r[...], b3r[...])
        cq_s[...] = cq
        qn_s[...] = jnp.sum(cq * cq, axis=1, keepdims=True)
        tv_s[...] = jnp.full(tv_s.shape, jnp.inf, jnp.float32)
        ti_s[...] = jnp.zeros(ti_s.shape, jnp.int32)

    ck = _mlp3_relu(keys_ref[...], W1r[...], b1r[...], W2r[...], b2r[...],
                    W3r[...], b3r[...])
    kn = jnp.sum(ck * ck, axis=1)
    qk = lax.dot_general(cq_s[...], ck, (((1,), (1,)), ((), ())),
                         preferred_element_type=jnp.float32)
    db_s[...] = qn_s[...] + kn[None, :] - 2.0 * qk

    _stream_merge(db_s, t * jnp.int32(kt), tv_s, ti_s)

    nv = tv_s[...]
    w = 1.0 / (nv + 1e-6)
    w_out[...] = w / jnp.sum(w, axis=1, keepdims=True)
    idx_out[...] = ti_s[...]
    conf_out[...] = 1.0 / (nv[:, :1] + 1e-6)


def _topk_search(query, mem_keys, cW1, cb1, cW2, cb2, cW3, cb3, kt):
    Q, D = query.shape
    K = mem_keys.shape[0]
    C = cW3.shape[1]
    assert K % kt == 0
    num_tiles = K // kt
    const = lambda *_: (0, 0)
    grid_spec = pltpu.PrefetchScalarGridSpec(
        num_scalar_prefetch=0,
        grid=(num_tiles,),
        in_specs=[
            pl.BlockSpec((Q, D), const),
            pl.BlockSpec((kt, D), lambda t: (t, 0)),
            pl.BlockSpec(cW1.shape, const),
            pl.BlockSpec((1, cb1.shape[-1]), const),
            pl.BlockSpec(cW2.shape, const),
            pl.BlockSpec((1, cb2.shape[-1]), const),
            pl.BlockSpec(cW3.shape, const),
            pl.BlockSpec((1, cb3.shape[-1]), const),
        ],
        out_specs=[
            pl.BlockSpec((Q, KSEL), const),
            pl.BlockSpec((Q, KSEL), const),
            pl.BlockSpec((Q, 1), const),
        ],
        scratch_shapes=[
            pltpu.VMEM((Q, C), jnp.float32),
            pltpu.VMEM((Q, 1), jnp.float32),
            pltpu.VMEM((Q, KSEL), jnp.float32),
            pltpu.VMEM((Q, KSEL), jnp.int32),
            pltpu.VMEM((Q, kt), jnp.float32),
        ],
    )
    return pl.pallas_call(
        functools.partial(_topk_body, num_tiles, kt),
        grid_spec=grid_spec,
        out_shape=[
            jax.ShapeDtypeStruct((Q, KSEL), jnp.int32),
            jax.ShapeDtypeStruct((Q, KSEL), jnp.float32),
            jax.ShapeDtypeStruct((Q, 1), jnp.float32),
        ],
        compiler_params=pltpu.CompilerParams(
            dimension_semantics=("arbitrary",)),
    )(query, mem_keys, cW1, cb1.reshape(1, -1), cW2, cb2.reshape(1, -1),
      cW3, cb3.reshape(1, -1))


def _sc_gather(table, idx2d):
    V, C = table.shape
    R, CH = idx2d.shape
    B = R * CH
    info = plsc.get_sparse_core_info()
    NW = info.num_cores * info.num_subcores
    rows_per_w = R // NW
    b_per_w = rows_per_w * CH
    mesh = plsc.VectorSubcoreMesh(core_axis_name="c", subcore_axis_name="s")

    @functools.partial(
        pl.kernel, mesh=mesh,
        out_type=jax.ShapeDtypeStruct((B, C), jnp.float32),
        scratch_types=[
            pltpu.VMEM((rows_per_w, CH), jnp.int32),
            pltpu.VMEM((b_per_w, C), jnp.float32),
            pltpu.SemaphoreType.DMA,
        ],
    )
    def kfn(table_hbm, idx_hbm, out_hbm, idx_v, rows_v, sem):
        wid = lax.axis_index("s") * info.num_cores + lax.axis_index("c")
        pltpu.sync_copy(idx_hbm.at[pl.ds(wid * rows_per_w, rows_per_w)], idx_v)
        copies = []
        for c in range(rows_per_w):
            copies.append(pltpu.async_copy(
                table_hbm.at[idx_v.at[c]],
                rows_v.at[pl.ds(c * CH, CH)], sem))
        for cp in copies:
            cp.wait()
        pltpu.sync_copy(rows_v, out_hbm.at[pl.ds(wid * b_per_w, b_per_w)])

    return kfn(table, idx2d)


def _combine_body(g_ref, w_ref, W1r, b1r, W2r, b2r, W3r, b3r, out_ref):
    w = w_ref[...]
    acc = g_ref[:, 0, :] * w[:, 0:1]
    for j in range(1, KSEL):
        acc = acc + g_ref[:, j, :] * w[:, j:j + 1]
    out_ref[...] = _mlp3_relu(acc, W1r[...], b1r[...], W2r[...], b2r[...],
                              W3r[...], b3r[...])


def _combine_decompress(gathered3, weights, dW1, db1, dW2, db2, dW3, db3):
    Q, ks, C = gathered3.shape
    D = dW3.shape[1]
    return pl.pallas_call(
        _combine_body,
        out_shape=jax.ShapeDtypeStruct((Q, D), jnp.float32),
    )(gathered3, weights, dW1, db1.reshape(1, -1), dW2, db2.reshape(1, -1),
      dW3, db3.reshape(1, -1))


def kernel(query, mem_keys, mem_values,
           cW1, cb1, cW2, cb2, cW3, cb3,
           dW1, db1, dW2, db2, dW3, db3, k):
    Q = query.shape[0]
    C = mem_values.shape[1]
    idx, weights, conf = _topk_search(query, mem_keys,
                                      cW1, cb1, cW2, cb2, cW3, cb3, kt=2000)
    idx2d = idx.reshape(-1, 128)
    gathered = _sc_gather(mem_values, idx2d)
    retrieved = _combine_decompress(gathered.reshape(Q, KSEL, C), weights,
                                    dW1, db1, dW2, db2, dW3, db3)
    return retrieved, conf.reshape(Q)


# speedup vs baseline: 1.8785x; 1.0267x over previous
import functools

import jax
import jax.numpy as jnp
from jax import lax
from jax.experimental import pallas as pl
from jax.experimental.pallas import tpu as pltpu
from jax.experimental.pallas import tpu_sc as plsc

KSEL = 16


def _mlp3_relu(x, W1, b1, W2, b2, W3, b3):
    h = jnp.maximum(x @ W1 + b1, 0.0)
    h = jnp.maximum(h @ W2 + b2, 0.0)
    return h @ W3 + b3


def _stream_merge(dbuf, col0, tv_s, ti_s, done_s):
    Q, kt = dbuf.shape
    big_idx = jnp.int32(2 ** 30)

    def one_pass():
        d = dbuf[...]
        gidx = lax.broadcasted_iota(jnp.int32, (Q, kt), 1) + col0
        m = jnp.min(d, axis=1, keepdims=True)
        sel = jnp.min(jnp.where(d == m, gidx, big_idx), axis=1,
                      keepdims=True)
        dbuf[...] = jnp.where(gidx == sel, jnp.inf, d)
        tv = tv_s[...]
        ti = ti_s[...]
        thr = tv[:, KSEL - 1:KSEL]
        tvm1 = jnp.concatenate(
            [jnp.full((Q, 1), -jnp.inf, jnp.float32), tv[:, :KSEL - 1]], 1)
        tim1 = jnp.concatenate(
            [jnp.zeros((Q, 1), jnp.int32), ti[:, :KSEL - 1]], 1)
        stay = tv <= m
        ins = jnp.logical_and(jnp.logical_not(stay), tvm1 <= m)
        tv_s[...] = jnp.where(stay, tv, jnp.where(ins, m, tvm1))
        ti_s[...] = jnp.where(stay, ti, jnp.where(ins, sel, tim1))
        return jnp.max(jnp.where(m < thr, 1, 0))

    done_s[0] = 0
    BLK = 2
    for _ in range((KSEL + 1 + BLK - 1) // BLK):
        @pl.when(done_s[0] == 0)
        def _blk():
            cont = 0
            for _ in range(BLK):
                cont = one_pass()
            done_s[0] = 1 - cont


def _topk_body(num_tiles, kt,
               q_ref, keys_ref, W1r, b1r, W2r, b2r, W3r, b3r,
               idx_out, w_out, conf_out,
               cq_s, qn_s, tv_s, ti_s, db_s, done_s):
    t = pl.program_id(0)

    @pl.when(t == 0)
    def _init():
        cq = _mlp3_relu(q_ref[...], W1r[...], b1r[...], W2r[...], b2r[...],
                        W3r[...], b3r[...])
        cq_s[...] = cq
        qn_s[...] = jnp.sum(cq * cq, axis=1, keepdims=True)
        tv_s[...] = jnp.full(tv_s.shape, jnp.inf, jnp.float32)
        ti_s[...] = jnp.zeros(ti_s.shape, jnp.int32)

    ck = _mlp3_relu(keys_ref[...], W1r[...], b1r[...], W2r[...], b2r[...],
                    W3r[...], b3r[...])
    kn = jnp.sum(ck * ck, axis=1)
    qk = lax.dot_general(cq_s[...], ck, (((1,), (1,)), ((), ())),
                         preferred_element_type=jnp.float32)
    db_s[...] = qn_s[...] + kn[None, :] - 2.0 * qk

    _stream_merge(db_s, t * jnp.int32(kt), tv_s, ti_s, done_s)

    nv = tv_s[...]
    w = 1.0 / (nv + 1e-6)
    w_out[...] = w / jnp.sum(w, axis=1, keepdims=True)
    idx_out[...] = ti_s[...]
    conf_out[...] = 1.0 / (nv[:, :1] + 1e-6)


def _topk_search(query, mem_keys, cW1, cb1, cW2, cb2, cW3, cb3, kt):
    Q, D = query.shape
    K = mem_keys.shape[0]
    C = cW3.shape[1]
    assert K % kt == 0
    num_tiles = K // kt
    const = lambda *_: (0, 0)
    grid_spec = pltpu.PrefetchScalarGridSpec(
        num_scalar_prefetch=0,
        grid=(num_tiles,),
        in_specs=[
            pl.BlockSpec((Q, D), const),
            pl.BlockSpec((kt, D), lambda t: (t, 0)),
            pl.BlockSpec(cW1.shape, const),
            pl.BlockSpec((1, cb1.shape[-1]), const),
            pl.BlockSpec(cW2.shape, const),
            pl.BlockSpec((1, cb2.shape[-1]), const),
            pl.BlockSpec(cW3.shape, const),
            pl.BlockSpec((1, cb3.shape[-1]), const),
        ],
        out_specs=[
            pl.BlockSpec((Q, KSEL), const),
            pl.BlockSpec((Q, KSEL), const),
            pl.BlockSpec((Q, 1), const),
        ],
        scratch_shapes=[
            pltpu.VMEM((Q, C), jnp.float32),
            pltpu.VMEM((Q, 1), jnp.float32),
            pltpu.VMEM((Q, KSEL), jnp.float32),
            pltpu.VMEM((Q, KSEL), jnp.int32),
            pltpu.VMEM((Q, kt), jnp.float32),
            pltpu.SMEM((1,), jnp.int32),
        ],
    )
    return pl.pallas_call(
        functools.partial(_topk_body, num_tiles, kt),
        grid_spec=grid_spec,
        out_shape=[
            jax.ShapeDtypeStruct((Q, KSEL), jnp.int32),
            jax.ShapeDtypeStruct((Q, KSEL), jnp.float32),
            jax.ShapeDtypeStruct((Q, 1), jnp.float32),
        ],
        compiler_params=pltpu.CompilerParams(
            dimension_semantics=("arbitrary",)),
    )(query, mem_keys, cW1, cb1.reshape(1, -1), cW2, cb2.reshape(1, -1),
      cW3, cb3.reshape(1, -1))


def _sc_gather(table, idx2d):
    V, C = table.shape
    R, CH = idx2d.shape
    B = R * CH
    info = plsc.get_sparse_core_info()
    NW = info.num_cores * info.num_subcores
    rows_per_w = R // NW
    b_per_w = rows_per_w * CH
    mesh = plsc.VectorSubcoreMesh(core_axis_name="c", subcore_axis_name="s")

    @functools.partial(
        pl.kernel, mesh=mesh,
        out_type=jax.ShapeDtypeStruct((B, C), jnp.float32),
        scratch_types=[
            pltpu.VMEM((rows_per_w, CH), jnp.int32),
            pltpu.VMEM((b_per_w, C), jnp.float32),
            pltpu.SemaphoreType.DMA,
        ],
    )
    def kfn(table_hbm, idx_hbm, out_hbm, idx_v, rows_v, sem):
        wid = lax.axis_index("s") * info.num_cores + lax.axis_index("c")
        pltpu.sync_copy(idx_hbm.at[pl.ds(wid * rows_per_w, rows_per_w)], idx_v)
        copies = []
        for c in range(rows_per_w):
            copies.append(pltpu.async_copy(
                table_hbm.at[idx_v.at[c]],
                rows_v.at[pl.ds(c * CH, CH)], sem))
        for cp in copies:
            cp.wait()
        pltpu.sync_copy(rows_v, out_hbm.at[pl.ds(wid * b_per_w, b_per_w)])

    return kfn(table, idx2d)


def _combine_body(g_ref, w_ref, W1r, b1r, W2r, b2r, W3r, b3r, out_ref):
    w = w_ref[...]
    acc = g_ref[:, 0, :] * w[:, 0:1]
    for j in range(1, KSEL):
        acc = acc + g_ref[:, j, :] * w[:, j:j + 1]
    out_ref[...] = _mlp3_relu(acc, W1r[...], b1r[...], W2r[...], b2r[...],
                              W3r[...], b3r[...])


def _combine_decompress(gathered3, weights, dW1, db1, dW2, db2, dW3, db3):
    Q, ks, C = gathered3.shape
    D = dW3.shape[1]
    return pl.pallas_call(
        _combine_body,
        out_shape=jax.ShapeDtypeStruct((Q, D), jnp.float32),
    )(gathered3, weights, dW1, db1.reshape(1, -1), dW2, db2.reshape(1, -1),
      dW3, db3.reshape(1, -1))


def kernel(query, mem_keys, mem_values,
           cW1, cb1, cW2, cb2, cW3, cb3,
           dW1, db1, dW2, db2, dW3, db3, k):
    Q = query.shape[0]
    C = mem_values.shape[1]
    idx, weights, conf = _topk_search(query, mem_keys,
                                      cW1, cb1, cW2, cb2, cW3, cb3, kt=2000)
    idx2d = idx.reshape(-1, 128)
    gathered = _sc_gather(mem_values, idx2d)
    retrieved = _combine_decompress(gathered.reshape(Q, KSEL, C), weights,
                                    dW1, db1, dW2, db2, dW3, db3)
    return retrieved, conf.reshape(Q)
